# baseline (speedup 1.0000x reference)
import functools

import jax
import jax.numpy as jnp
from jax.experimental import pallas as pl
from jax.experimental.pallas import tpu as pltpu

THRESH = 0.125


def _moe_body(x_ref, gw_ref, gb_ref, ews_ref, ebs_ref, o_ref, xs_scr):
    x = x_ref[...]
    logits = jnp.dot(x, gw_ref[...],
                     preferred_element_type=jnp.float32) + gb_ref[...]
    probs = jax.nn.softmax(logits, axis=-1)
    w = jnp.where(probs >= THRESH, probs, 0.0)
    s = jnp.sum(w, axis=-1, keepdims=True)
    s = jnp.where(s == 0.0, 1.0, s)
    w = w / s
    D = x.shape[1]
    E = w.shape[1]
    for e in range(E):
        xs_scr[:, e * D:(e + 1) * D] = (w[:, e:e + 1] * x).astype(jnp.bfloat16)
    y = jnp.dot(xs_scr[...], ews_ref[...], preferred_element_type=jnp.float32)
    o_ref[...] = y + jnp.dot(w, ebs_ref[...],
                             preferred_element_type=jnp.float32)


@functools.partial(jax.jit, static_argnums=())
def _moe(x, gate_W, gate_b2, expert_Ws, expert_b):
    T, D = x.shape
    E = gate_W.shape[-1]
    BT = 512
    grid = (T // BT,)
    return pl.pallas_call(
        _moe_body,
        grid=grid,
        in_specs=[
            pl.BlockSpec((BT, D), lambda t: (t, 0)),
            pl.BlockSpec((D, E), lambda t: (0, 0)),
            pl.BlockSpec((1, E), lambda t: (0, 0)),
            pl.BlockSpec((E * D, D), lambda t: (0, 0)),
            pl.BlockSpec((E, D), lambda t: (0, 0)),
        ],
        out_specs=pl.BlockSpec((BT, D), lambda t: (t, 0)),
        out_shape=jax.ShapeDtypeStruct((T, D), jnp.float32),
        scratch_shapes=[pltpu.VMEM((BT, E * D), jnp.bfloat16)],
    )(x, gate_W, gate_b2, expert_Ws, expert_b)


def kernel(inputs, patch_h, patch_w, gate_W, gate_b, expert_W, expert_b):
    x = inputs.reshape((-1, inputs.shape[-1]))
    E, D = expert_b.shape
    expert_Ws = expert_W.astype(jnp.bfloat16).reshape(E * D, D)
    out = _moe(x, gate_W, gate_b.reshape(1, -1), expert_Ws, expert_b)
    return out.reshape(inputs.shape[:-1] + (out.shape[-1],))


# speedup vs baseline: 1.0690x; 1.0690x over previous
import functools

import jax
import jax.numpy as jnp
from jax.experimental import pallas as pl
from jax.experimental.pallas import tpu as pltpu

THRESH = 0.125


def _moe_body(x_ref, gw_ref, gb_ref, ews_ref, ebs_ref, o_ref, xs_scr, wb_scr):
    t = pl.program_id(0)

    @pl.when(t == 0)
    def _cast_w():
        wb_scr[...] = ews_ref[...].astype(jnp.bfloat16)

    x = x_ref[...]
    logits = jnp.dot(x, gw_ref[...],
                     preferred_element_type=jnp.float32) + gb_ref[...]
    probs = jax.nn.softmax(logits, axis=-1)
    w = jnp.where(probs >= THRESH, probs, 0.0)
    s = jnp.sum(w, axis=-1, keepdims=True)
    s = jnp.where(s == 0.0, 1.0, s)
    w = w / s
    D = x.shape[1]
    E = w.shape[1]
    for e in range(E):
        xs_scr[:, e * D:(e + 1) * D] = (w[:, e:e + 1] * x).astype(jnp.bfloat16)
    y = jnp.dot(xs_scr[...], wb_scr[...], preferred_element_type=jnp.float32)
    o_ref[...] = y + jnp.dot(w, ebs_ref[...],
                             preferred_element_type=jnp.float32)


@functools.partial(jax.jit, static_argnums=())
def _moe(x, gate_W, gate_b2, expert_Ws, expert_b):
    T, D = x.shape
    E = gate_W.shape[-1]
    BT = 256
    grid = (T // BT,)
    return pl.pallas_call(
        _moe_body,
        grid=grid,
        in_specs=[
            pl.BlockSpec((BT, D), lambda t: (t, 0)),
            pl.BlockSpec((D, E), lambda t: (0, 0)),
            pl.BlockSpec((1, E), lambda t: (0, 0)),
            pl.BlockSpec((E * D, D), lambda t: (0, 0)),
            pl.BlockSpec((E, D), lambda t: (0, 0)),
        ],
        out_specs=pl.BlockSpec((BT, D), lambda t: (t, 0)),
        out_shape=jax.ShapeDtypeStruct((T, D), jnp.float32),
        scratch_shapes=[pltpu.VMEM((BT, E * D), jnp.bfloat16),
                        pltpu.VMEM((E * D, D), jnp.bfloat16)],
    )(x, gate_W, gate_b2, expert_Ws, expert_b)


def kernel(inputs, patch_h, patch_w, gate_W, gate_b, expert_W, expert_b):
    x = inputs.reshape((-1, inputs.shape[-1]))
    E, D = expert_b.shape
    out = _moe(x, gate_W, gate_b.reshape(1, -1), expert_W.reshape(E * D, D),
               expert_b)
    return out.reshape(inputs.shape[:-1] + (out.shape[-1],))
